# baseline (speedup 1.0000x reference)
import functools

import jax
import jax.numpy as jnp
from jax import lax
from jax.experimental import pallas as pl
from jax.experimental.pallas import tpu as pltpu
from jax.experimental.pallas import tpu_sc as plsc

B, T, H = 1024, 50, 128
BT = B * T
KV = 1000
OV = 1000
SV = 100000
NC, NS = 2, 16
NW = NC * NS
ROWS_W = BT // NW
CHUNK = 80
NCH = ROWS_W // CHUNK
SROWS = B // NW
NG = 6

KW = 16 * H
OW = 8 * H
SW = 8 * H
TW = 8 * H


def _sc_gather(cat_f, scat_f, k_tab, o_tab, s_tab):
    mesh = plsc.VectorSubcoreMesh(core_axis_name="c", subcore_axis_name="s")

    @functools.partial(
        pl.kernel,
        out_type=(
            jax.ShapeDtypeStruct((BT, KW), jnp.float32),
            jax.ShapeDtypeStruct((BT, OW), jnp.float32),
            jax.ShapeDtypeStruct((B, SW), jnp.float32),
        ),
        mesh=mesh,
        scratch_types=[
            pltpu.VMEM((NG, CHUNK), jnp.int32),
            pltpu.VMEM((NG, CHUNK, H), jnp.float32),
            pltpu.VMEM((SROWS,), jnp.int32),
            pltpu.VMEM((SROWS, H), jnp.float32),
            pltpu.SemaphoreType.DMA,
        ],
    )
    def body(cat_hbm, scat_hbm, ktab_hbm, otab_hbm, stab_hbm,
             kout_hbm, oout_hbm, sout_hbm,
             idx_v, rows_v, sidx_v, srows_v, sem):
        wid = lax.axis_index("s") * NC + lax.axis_index("c")

        sbase = wid * SROWS
        for i in range(2):
            pltpu.sync_copy(scat_hbm.at[pl.ds(i * B + sbase, SROWS)], sidx_v)
            if i:
                for v in range(SROWS // 16):
                    sl = pl.ds(v * 16, 16)
                    sidx_v[sl] = sidx_v[sl] + i * SV
            pltpu.async_copy(stab_hbm.at[sidx_v], srows_v, sem).wait()
            pltpu.sync_copy(
                srows_v, sout_hbm.at[pl.ds(sbase, SROWS), pl.ds(i * H, H)])

        def chunk_body(c, carry):
            base = wid * ROWS_W + c * CHUNK
            for g in range(NG):
                pltpu.sync_copy(
                    cat_hbm.at[pl.ds(g * BT + base, CHUNK)], idx_v.at[g])
            for g in range(NG):
                off = g * KV if g < 4 else (g - 4) * OV
                if off:
                    for v in range(CHUNK // 16):
                        sl = pl.ds(v * 16, 16)
                        idx_v[g, sl] = idx_v[g, sl] + off
            descs = []
            for g in range(NG):
                tab = ktab_hbm if g < 4 else otab_hbm
                descs.append(
                    pltpu.async_copy(tab.at[idx_v.at[g]], rows_v.at[g], sem))
            for d in descs:
                d.wait()
            descs = []
            for g in range(NG):
                if g < 4:
                    dst = kout_hbm.at[pl.ds(base, CHUNK), pl.ds(g * H, H)]
                else:
                    dst = oout_hbm.at[pl.ds(base, CHUNK), pl.ds((g - 4) * H, H)]
                descs.append(pltpu.async_copy(rows_v.at[g], dst, sem))
            for d in descs:
                d.wait()
            return carry

        lax.fori_loop(0, NCH, chunk_body, 0)

    return body(cat_f, scat_f, k_tab, o_tab, s_tab)


def _expand_linear(vec, bias):
    n = vec.shape[0]
    eye = jnp.eye(n, dtype=vec.dtype)
    m_top = eye[:, :, None] * vec[None, :, :]
    m = jnp.concatenate([m_top, bias[None, :, :]], axis=0)
    return m.reshape(n + 1, n * H)


def _mxu_body(c_ref, m_ref, alias_ref, out_ref):
    out_ref[...] = jnp.dot(c_ref[...], m_ref[...],
                           preferred_element_type=jnp.float32)


def _cont_fill(cont_ext, m, cat_buf, ncat, nv_step, rch):
    n_rows, ncp1 = cont_ext.shape
    ncont = ncp1 - 1
    nsteps = ncont // nv_step
    colblk = nv_step * H
    return pl.pallas_call(
        _mxu_body,
        grid=(n_rows // rch, nsteps),
        in_specs=[
            pl.BlockSpec((rch, ncp1), lambda i, j: (i, 0)),
            pl.BlockSpec((ncp1, colblk), lambda i, j: (0, j)),
            pl.BlockSpec((8, 128), lambda i, j: (0, 0)),
        ],
        out_specs=pl.BlockSpec(
            (rch, colblk), lambda i, j: (i, j + ncat // nv_step)),
        out_shape=jax.ShapeDtypeStruct(cat_buf.shape, jnp.float32),
        input_output_aliases={2: 0},
    )(cont_ext, m, cat_buf)


def _tgt_body(c_ref, vec_ref, bias_ref, out_ref):
    out_ref[...] = (c_ref[...] * vec_ref[...][0][None, :]
                    + bias_ref[...][0][None, :])


def _tgt_fill(cont2, vec, bias, rch):
    n_rows = cont2.shape[0]
    return pl.pallas_call(
        _tgt_body,
        grid=(n_rows // rch,),
        in_specs=[
            pl.BlockSpec((rch, 1), lambda i: (i, 0)),
            pl.BlockSpec((1, H), lambda i: (0, 0)),
            pl.BlockSpec((1, H), lambda i: (0, 0)),
        ],
        out_specs=pl.BlockSpec((rch, H), lambda i: (i, 0)),
        out_shape=jax.ShapeDtypeStruct((n_rows, TW), jnp.float32),
    )(cont2, vec, bias)


def kernel(s_cat, s_cont, k_cat, k_cont, o_cat, o_cont, target,
           s_cat_tables, k_cat_tables, o_cat_tables,
           s_cont_vec, s_cont_bias, k_cont_vec, k_cont_bias,
           o_cont_vec, o_cont_bias, tgt_vec, tgt_bias):
    kcat_t = k_cat.reshape(BT, 4).T.reshape(-1)
    ocat_t = o_cat.reshape(BT, 2).T.reshape(-1)
    cat_f = jnp.concatenate([kcat_t, ocat_t])
    scat_f = s_cat[:, 0, :].T.reshape(-1)
    k_tab = k_cat_tables.reshape(4 * KV, H)
    o_tab = o_cat_tables.reshape(2 * OV, H)
    s_tab = s_cat_tables.reshape(2 * SV, H)

    kbuf, obuf, sbuf = _sc_gather(cat_f, scat_f, k_tab, o_tab, s_tab)

    ones_bt = jnp.ones((BT, 1), jnp.float32)
    k_ext = jnp.concatenate([k_cont.reshape(BT, 8), ones_bt], axis=1)
    o_ext = jnp.concatenate([o_cont.reshape(BT, 6), ones_bt], axis=1)
    s_ext = jnp.concatenate(
        [s_cont[:, 0, :], jnp.ones((B, 1), jnp.float32)], axis=1)

    kbuf = _cont_fill(k_ext, _expand_linear(k_cont_vec, k_cont_bias),
                      kbuf, 4, 4, 1024)
    obuf = _cont_fill(o_ext, _expand_linear(o_cont_vec, o_cont_bias),
                      obuf, 2, 2, 1024)
    sbuf = _cont_fill(s_ext, _expand_linear(s_cont_vec, s_cont_bias),
                      sbuf, 2, 2, 512)
    tbuf = _tgt_fill(target.reshape(BT, 1), tgt_vec, tgt_bias, 2048)

    return (sbuf.reshape(B, 8, H)[:, :6, :],
            kbuf.reshape(B, T, 16, H)[:, :, :12, :],
            obuf.reshape(B, T, 8, H),
            tbuf.reshape(B, T, 8, H)[:, :, :1, :])


# speedup vs baseline: 1.5017x; 1.5017x over previous
import functools

import jax
import jax.numpy as jnp
from jax import lax
from jax.experimental import pallas as pl
from jax.experimental.pallas import tpu as pltpu
from jax.experimental.pallas import tpu_sc as plsc

B, T, H = 1024, 50, 128
BT = B * T
KV = 1000
OV = 1000
SV = 100000
NC, NS = 2, 16
NW = NC * NS
ROWS_W = BT // NW
CHUNK = 80
NCH = ROWS_W // CHUNK
SROWS = B // NW
NG = 6


def _sc_gather(cat_f, scat_f, k_tab, o_tab, s_tab):
    mesh = plsc.VectorSubcoreMesh(core_axis_name="c", subcore_axis_name="s")

    @functools.partial(
        pl.kernel,
        out_type=(
            jax.ShapeDtypeStruct((BT, 4 * H), jnp.float32),
            jax.ShapeDtypeStruct((BT, 8 * H), jnp.float32),
            jax.ShapeDtypeStruct((B, 2 * H), jnp.float32),
        ),
        mesh=mesh,
        scratch_types=[
            pltpu.VMEM((NG, CHUNK), jnp.int32),
            pltpu.VMEM((NG, CHUNK, H), jnp.float32),
            pltpu.VMEM((SROWS,), jnp.int32),
            pltpu.VMEM((SROWS, H), jnp.float32),
            pltpu.SemaphoreType.DMA,
        ],
    )
    def body(cat_hbm, scat_hbm, ktab_hbm, otab_hbm, stab_hbm,
             kout_hbm, oout_hbm, sout_hbm,
             idx_v, rows_v, sidx_v, srows_v, sem):
        wid = lax.axis_index("s") * NC + lax.axis_index("c")

        sbase = wid * SROWS
        for i in range(2):
            pltpu.sync_copy(scat_hbm.at[pl.ds(i * B + sbase, SROWS)], sidx_v)
            if i:
                for v in range(SROWS // 16):
                    sl = pl.ds(v * 16, 16)
                    sidx_v[sl] = sidx_v[sl] + i * SV
            pltpu.async_copy(stab_hbm.at[sidx_v], srows_v, sem).wait()
            pltpu.sync_copy(
                srows_v, sout_hbm.at[pl.ds(sbase, SROWS), pl.ds(i * H, H)])

        def chunk_body(c, carry):
            base = wid * ROWS_W + c * CHUNK
            for g in range(NG):
                pltpu.sync_copy(
                    cat_hbm.at[pl.ds(g * BT + base, CHUNK)], idx_v.at[g])
            for g in range(NG):
                off = g * KV if g < 4 else (g - 4) * OV
                if off:
                    for v in range(CHUNK // 16):
                        sl = pl.ds(v * 16, 16)
                        idx_v[g, sl] = idx_v[g, sl] + off
            descs = []
            for g in range(NG):
                tab = ktab_hbm if g < 4 else otab_hbm
                descs.append(
                    pltpu.async_copy(tab.at[idx_v.at[g]], rows_v.at[g], sem))
            for d in descs:
                d.wait()
            descs = []
            for g in range(NG):
                if g < 4:
                    dst = kout_hbm.at[pl.ds(base, CHUNK), pl.ds(g * H, H)]
                else:
                    dst = oout_hbm.at[pl.ds(base, CHUNK), pl.ds((g - 4) * H, H)]
                descs.append(pltpu.async_copy(rows_v.at[g], dst, sem))
            for d in descs:
                d.wait()
            return carry

        lax.fori_loop(0, NCH, chunk_body, 0)

    return body(cat_f, scat_f, k_tab, o_tab, s_tab)


def _expand_linear(vec, bias):
    n = vec.shape[0]
    eye = jnp.eye(n, dtype=vec.dtype)
    m_top = eye[:, :, None] * vec[None, :, :]
    m = jnp.concatenate([m_top, bias[None, :, :]], axis=0)
    return m.reshape(n + 1, n * H)


def _mxu_body(c_ref, m_ref, alias_ref, out_ref):
    out_ref[...] = jnp.dot(c_ref[...], m_ref[...],
                           preferred_element_type=jnp.float32)


def _cont_fill(cont_ext, m, cat_buf, ncat, nv_step, rch):
    n_rows, ncp1 = cont_ext.shape
    ncont = ncp1 - 1
    nsteps = ncont // nv_step
    colblk = nv_step * H
    return pl.pallas_call(
        _mxu_body,
        grid=(n_rows // rch, nsteps),
        in_specs=[
            pl.BlockSpec((rch, ncp1), lambda i, j: (i, 0)),
            pl.BlockSpec((ncp1, colblk), lambda i, j: (0, j)),
            pl.BlockSpec((8, 128), lambda i, j: (0, 0)),
        ],
        out_specs=pl.BlockSpec(
            (rch, colblk), lambda i, j: (i, j + ncat // nv_step)),
        out_shape=jax.ShapeDtypeStruct(cat_buf.shape, jnp.float32),
        input_output_aliases={2: 0},
    )(cont_ext, m, cat_buf)


GBK = 8
GBT = 32
SB = 256


def _known_body(cat_ref, c_ref, m_ref, out_ref):
    cont = jnp.dot(c_ref[...], m_ref[...],
                   preferred_element_type=jnp.float32)
    for v in range(4):
        out_ref[:, :, v, :] = (
            cat_ref[:, pl.ds(v * H, H)].reshape(GBK, T, H))
    for cv in range(8):
        out_ref[:, :, 4 + cv, :] = (
            cont[:, cv * H:(cv + 1) * H].reshape(GBK, T, H))


def _known_fill(cat_buf, cont_ext, m):
    return pl.pallas_call(
        _known_body,
        grid=(B // GBK,),
        in_specs=[
            pl.BlockSpec((GBK * T, 4 * H), lambda i: (i, 0)),
            pl.BlockSpec((GBK * T, 9), lambda i: (i, 0)),
            pl.BlockSpec((9, 8 * H), lambda i: (0, 0)),
        ],
        out_specs=pl.BlockSpec((GBK, T, 12, H), lambda i: (i, 0, 0, 0)),
        out_shape=jax.ShapeDtypeStruct((B, T, 12, H), jnp.float32),
    )(cat_buf, cont_ext, m)


def _static_body(cat_ref, c_ref, m_ref, out_ref):
    cont = jnp.dot(c_ref[...], m_ref[...],
                   preferred_element_type=jnp.float32)
    for v in range(2):
        out_ref[:, v, :] = cat_ref[:, pl.ds(v * H, H)]
    for cv in range(4):
        out_ref[:, 2 + cv, :] = cont[:, cv * H:(cv + 1) * H]


def _static_fill(cat_buf, cont_ext, m):
    return pl.pallas_call(
        _static_body,
        grid=(B // SB,),
        in_specs=[
            pl.BlockSpec((SB, 2 * H), lambda i: (i, 0)),
            pl.BlockSpec((SB, 5), lambda i: (i, 0)),
            pl.BlockSpec((5, 4 * H), lambda i: (0, 0)),
        ],
        out_specs=pl.BlockSpec((SB, 6, H), lambda i: (i, 0, 0)),
        out_shape=jax.ShapeDtypeStruct((B, 6, H), jnp.float32),
    )(cat_buf, cont_ext, m)


def _tgt_body(c_ref, vec_ref, bias_ref, out_ref):
    out_ref[:, :, 0, :] = (
        c_ref[...] * vec_ref[...][0][None, :] + bias_ref[...][0][None, :]
    ).reshape(GBT, T, H)


def _tgt_fill(cont2, vec, bias):
    return pl.pallas_call(
        _tgt_body,
        grid=(B // GBT,),
        in_specs=[
            pl.BlockSpec((GBT * T, 1), lambda i: (i, 0)),
            pl.BlockSpec((1, H), lambda i: (0, 0)),
            pl.BlockSpec((1, H), lambda i: (0, 0)),
        ],
        out_specs=pl.BlockSpec((GBT, T, 1, H), lambda i: (i, 0, 0, 0)),
        out_shape=jax.ShapeDtypeStruct((B, T, 1, H), jnp.float32),
    )(cont2, vec, bias)


def kernel(s_cat, s_cont, k_cat, k_cont, o_cat, o_cont, target,
           s_cat_tables, k_cat_tables, o_cat_tables,
           s_cont_vec, s_cont_bias, k_cont_vec, k_cont_bias,
           o_cont_vec, o_cont_bias, tgt_vec, tgt_bias):
    kcat_t = k_cat.reshape(BT, 4).T.reshape(-1)
    ocat_t = o_cat.reshape(BT, 2).T.reshape(-1)
    cat_f = jnp.concatenate([kcat_t, ocat_t])
    scat_f = s_cat[:, 0, :].T.reshape(-1)
    k_tab = k_cat_tables.reshape(4 * KV, H)
    o_tab = o_cat_tables.reshape(2 * OV, H)
    s_tab = s_cat_tables.reshape(2 * SV, H)

    kcat_buf, obuf, scat_buf = _sc_gather(cat_f, scat_f, k_tab, o_tab, s_tab)

    ones_bt = jnp.ones((BT, 1), jnp.float32)
    k_ext = jnp.concatenate([k_cont.reshape(BT, 8), ones_bt], axis=1)
    o_ext = jnp.concatenate([o_cont.reshape(BT, 6), ones_bt], axis=1)
    s_ext = jnp.concatenate(
        [s_cont[:, 0, :], jnp.ones((B, 1), jnp.float32)], axis=1)

    k_full = _known_fill(kcat_buf, k_ext,
                         _expand_linear(k_cont_vec, k_cont_bias))
    obuf = _cont_fill(o_ext, _expand_linear(o_cont_vec, o_cont_bias),
                      obuf, 2, 2, 1024)
    s_full = _static_fill(scat_buf, s_ext,
                          _expand_linear(s_cont_vec, s_cont_bias))
    t_full = _tgt_fill(target.reshape(BT, 1), tgt_vec, tgt_bias)

    return (s_full,
            k_full,
            obuf.reshape(B, T, 8, H),
            t_full)


# speedup vs baseline: 1.9682x; 1.3106x over previous
import functools

import jax
import jax.numpy as jnp
from jax import lax
from jax.experimental import pallas as pl
from jax.experimental.pallas import tpu as pltpu
from jax.experimental.pallas import tpu_sc as plsc

B, T, H = 1024, 50, 128
BT = B * T
KV = 1000
OV = 1000
SV = 100000
NC, NS = 2, 16
NW = NC * NS

KCH = 64
KNCH = (4 * T * B) // KCH // NW
KSLOT = 5
OCH = 64
ROWS_W = BT // NW
ONCH = ROWS_W // OCH
SROWS = B // NW


def _sc_gather(kcat_f, ocat_f, scat_f, k_tab, o_tab, s_tab):
    mesh = plsc.VectorSubcoreMesh(core_axis_name="c", subcore_axis_name="s")

    @functools.partial(
        pl.kernel,
        out_type=(
            jax.ShapeDtypeStruct((T * 12 * B, H), jnp.float32),
            jax.ShapeDtypeStruct((BT, 8 * H), jnp.float32),
            jax.ShapeDtypeStruct((6 * B, H), jnp.float32),
        ),
        mesh=mesh,
        scratch_types=[
            pltpu.VMEM((KSLOT, KCH), jnp.int32),
            pltpu.VMEM((KSLOT, KCH, H), jnp.float32),
            pltpu.VMEM((2, OCH), jnp.int32),
            pltpu.VMEM((2, OCH, H), jnp.float32),
            pltpu.VMEM((SROWS,), jnp.int32),
            pltpu.VMEM((SROWS, H), jnp.float32),
            pltpu.SemaphoreType.DMA,
        ],
    )
    def body(kcat_hbm, ocat_hbm, scat_hbm, ktab_hbm, otab_hbm, stab_hbm,
             kout_hbm, oout_hbm, sout_hbm,
             kidx_v, krows_v, oidx_v, orows_v, sidx_v, srows_v, sem):
        wid = lax.axis_index("s") * NC + lax.axis_index("c")

        sbase = wid * SROWS
        for i in range(2):
            pltpu.sync_copy(scat_hbm.at[pl.ds(i * B + sbase, SROWS)], sidx_v)
            if i:
                for v in range(SROWS // 16):
                    sl = pl.ds(v * 16, 16)
                    sidx_v[sl] = sidx_v[sl] + i * SV
            pltpu.async_copy(stab_hbm.at[sidx_v], srows_v, sem).wait()
            pltpu.sync_copy(srows_v, sout_hbm.at[pl.ds(i * B + sbase, SROWS)])

        kv = wid // (NW // 4)
        koff = kv * KV
        rbase = (wid % (NW // 4)) * KNCH

        def kblock(jb, carry):
            c0 = jb * KSLOT
            for u in range(KSLOT):
                src = (wid * KNCH + c0 + u) * KCH
                pltpu.sync_copy(kcat_hbm.at[pl.ds(src, KCH)], kidx_v.at[u])
                for q in range(KCH // 16):
                    sl = pl.ds(q * 16, 16)
                    kidx_v[u, sl] = kidx_v[u, sl] + koff
            descs = [
                pltpu.async_copy(ktab_hbm.at[kidx_v.at[u]], krows_v.at[u], sem)
                for u in range(KSLOT)
            ]
            for d in descs:
                d.wait()
            for u in range(KSLOT):
                r = rbase + c0 + u
                t = r // (B // KCH)
                bc = r % (B // KCH)
                dst = (t * 12 + kv) * B + bc * KCH
                pltpu.sync_copy(krows_v.at[u], kout_hbm.at[pl.ds(dst, KCH)])
            return carry

        lax.fori_loop(0, KNCH // KSLOT, kblock, 0)

        def oblock(c, carry):
            base = wid * ROWS_W + c * OCH
            for g in range(2):
                pltpu.sync_copy(
                    ocat_hbm.at[pl.ds(g * BT + base, OCH)], oidx_v.at[g])
            for q in range(OCH // 16):
                sl = pl.ds(q * 16, 16)
                oidx_v[1, sl] = oidx_v[1, sl] + OV
            descs = [
                pltpu.async_copy(otab_hbm.at[oidx_v.at[g]], orows_v.at[g], sem)
                for g in range(2)
            ]
            for d in descs:
                d.wait()
            descs = [
                pltpu.async_copy(
                    orows_v.at[g],
                    oout_hbm.at[pl.ds(base, OCH), pl.ds(g * H, H)], sem)
                for g in range(2)
            ]
            for d in descs:
                d.wait()
            return carry

        lax.fori_loop(0, ONCH, oblock, 0)

    return body(kcat_f, ocat_f, scat_f, k_tab, o_tab, s_tab)


def _known_cont_body(c_ref, vec_ref, bias_ref, alias_ref, out_ref):
    j = pl.program_id(1)
    c = c_ref[0]
    vec = vec_ref[...]
    bias = bias_ref[...]
    for jj in range(8):

        @pl.when(j == jj)
        def _():
            out_ref[...] = jnp.dot(
                c[:, jj:jj + 1], vec[jj:jj + 1, :],
                preferred_element_type=jnp.float32) + bias[jj:jj + 1, :]


def _known_cont_fill(cont_tb, vec, bias, cat_buf):
    return pl.pallas_call(
        _known_cont_body,
        grid=(T, 8),
        in_specs=[
            pl.BlockSpec((1, B, 8), lambda t, j: (t, 0, 0)),
            pl.BlockSpec((8, H), lambda t, j: (0, 0)),
            pl.BlockSpec((8, H), lambda t, j: (0, 0)),
            pl.BlockSpec((8, H), lambda t, j: (0, 0)),
        ],
        out_specs=pl.BlockSpec((B, H), lambda t, j: (t * 12 + 4 + j, 0)),
        out_shape=jax.ShapeDtypeStruct((T * 12 * B, H), jnp.float32),
        input_output_aliases={3: 0},
    )(cont_tb, vec, bias, cat_buf)


OR_CH = 2048


def _obs_cont_body(c_ref, vec_ref, bias_ref, alias_ref, out_ref):
    j = pl.program_id(1)
    c = c_ref[...]
    vec = vec_ref[...]
    bias = bias_ref[...]
    for jj in range(3):

        @pl.when(j == jj)
        def _():
            for u in range(2):
                cv = 2 * jj + u
                out_ref[:, u * H:(u + 1) * H] = jnp.dot(
                    c[:, cv:cv + 1], vec[cv:cv + 1, :],
                    preferred_element_type=jnp.float32) + bias[cv:cv + 1, :]


def _obs_cont_fill(cont2, vec, bias, cat_buf):
    return pl.pallas_call(
        _obs_cont_body,
        grid=(BT // OR_CH, 3),
        in_specs=[
            pl.BlockSpec((OR_CH, 6), lambda i, j: (i, 0)),
            pl.BlockSpec((6, H), lambda i, j: (0, 0)),
            pl.BlockSpec((6, H), lambda i, j: (0, 0)),
            pl.BlockSpec((8, 128), lambda i, j: (0, 0)),
        ],
        out_specs=pl.BlockSpec((OR_CH, 2 * H), lambda i, j: (i, j + 1)),
        out_shape=jax.ShapeDtypeStruct((BT, 8 * H), jnp.float32),
        input_output_aliases={3: 0},
    )(cont2, vec, bias, cat_buf)


def _static_cont_body(c_ref, vec_ref, bias_ref, alias_ref, out_ref):
    v = pl.program_id(0)
    c = c_ref[...]
    vec = vec_ref[...]
    bias = bias_ref[...]
    for vv in range(4):

        @pl.when(v == vv)
        def _():
            out_ref[...] = jnp.dot(
                c[:, vv:vv + 1], vec[vv:vv + 1, :],
                preferred_element_type=jnp.float32) + bias[vv:vv + 1, :]


def _static_cont_fill(cont2, vec, bias, cat_buf):
    return pl.pallas_call(
        _static_cont_body,
        grid=(4,),
        in_specs=[
            pl.BlockSpec((B, 4), lambda v: (0, 0)),
            pl.BlockSpec((4, H), lambda v: (0, 0)),
            pl.BlockSpec((4, H), lambda v: (0, 0)),
            pl.BlockSpec((8, H), lambda v: (0, 0)),
        ],
        out_specs=pl.BlockSpec((B, H), lambda v: (v + 2, 0)),
        out_shape=jax.ShapeDtypeStruct((6 * B, H), jnp.float32),
        input_output_aliases={3: 0},
    )(cont2, vec, bias, cat_buf)


GBT = 32


def _tgt_body(c_ref, vec_ref, bias_ref, out_ref):
    out_ref[:, :, 0, :] = (
        c_ref[...] * vec_ref[...][0][None, :] + bias_ref[...][0][None, :]
    ).reshape(GBT, T, H)


def _tgt_fill(cont2, vec, bias):
    return pl.pallas_call(
        _tgt_body,
        grid=(B // GBT,),
        in_specs=[
            pl.BlockSpec((GBT * T, 1), lambda i: (i, 0)),
            pl.BlockSpec((1, H), lambda i: (0, 0)),
            pl.BlockSpec((1, H), lambda i: (0, 0)),
        ],
        out_specs=pl.BlockSpec((GBT, T, 1, H), lambda i: (i, 0, 0, 0)),
        out_shape=jax.ShapeDtypeStruct((B, T, 1, H), jnp.float32),
    )(cont2, vec, bias)


def kernel(s_cat, s_cont, k_cat, k_cont, o_cat, o_cont, target,
           s_cat_tables, k_cat_tables, o_cat_tables,
           s_cont_vec, s_cont_bias, k_cont_vec, k_cont_bias,
           o_cont_vec, o_cont_bias, tgt_vec, tgt_bias):
    kcat_f = jnp.transpose(k_cat, (2, 1, 0)).reshape(-1)
    ocat_f = o_cat.reshape(BT, 2).T.reshape(-1)
    scat_f = s_cat[:, 0, :].T.reshape(-1)
    k_tab = k_cat_tables.reshape(4 * KV, H)
    o_tab = o_cat_tables.reshape(2 * OV, H)
    s_tab = s_cat_tables.reshape(2 * SV, H)

    kbuf, obuf, sbuf = _sc_gather(kcat_f, ocat_f, scat_f, k_tab, o_tab, s_tab)

    kbuf = _known_cont_fill(jnp.transpose(k_cont, (1, 0, 2)),
                            k_cont_vec, k_cont_bias, kbuf)
    obuf = _obs_cont_fill(o_cont.reshape(BT, 6),
                          o_cont_vec, o_cont_bias, obuf)
    sbuf = _static_cont_fill(s_cont[:, 0, :],
                             s_cont_vec, s_cont_bias, sbuf)
    t_full = _tgt_fill(target.reshape(BT, 1), tgt_vec, tgt_bias)

    k_full = jnp.transpose(kbuf.reshape(T, 12, B, H), (2, 0, 1, 3))
    o_full = obuf.reshape(B, T, 8, H)
    s_full = jnp.transpose(sbuf.reshape(6, B, H), (1, 0, 2))
    return (s_full, k_full, o_full, t_full)


# speedup vs baseline: 2.3955x; 1.2171x over previous
import functools

import jax
import jax.numpy as jnp
from jax import lax
from jax.experimental import pallas as pl
from jax.experimental.pallas import tpu as pltpu
from jax.experimental.pallas import tpu_sc as plsc

B, T, H = 1024, 50, 128
BT = B * T
KV = 1000
OV = 1000
SV = 100000
NC, NS = 2, 16
NW = NC * NS

KCH = 64
KNCH = (4 * T * B) // KCH // NW
KSLOT = 5
OCH = 64
ROWS_W = BT // NW
ONCH = ROWS_W // OCH
SROWS = B // NW


def _sc_gather(kcat_f, ocat_f, scat_f, k_tab, o_tab, s_tab):
    mesh = plsc.VectorSubcoreMesh(core_axis_name="c", subcore_axis_name="s")

    @functools.partial(
        pl.kernel,
        out_type=(
            jax.ShapeDtypeStruct((T * 12 * B, H), jnp.float32),
            jax.ShapeDtypeStruct((BT, 8 * H), jnp.float32),
            jax.ShapeDtypeStruct((6 * B, H), jnp.float32),
        ),
        mesh=mesh,
        scratch_types=[
            pltpu.VMEM((KSLOT, KCH), jnp.int32),
            pltpu.VMEM((KSLOT, KCH, H), jnp.float32),
            pltpu.VMEM((2, OCH), jnp.int32),
            pltpu.VMEM((2, OCH, H), jnp.float32),
            pltpu.VMEM((SROWS,), jnp.int32),
            pltpu.VMEM((SROWS, H), jnp.float32),
            pltpu.SemaphoreType.DMA,
        ],
    )
    def body(kcat_hbm, ocat_hbm, scat_hbm, ktab_hbm, otab_hbm, stab_hbm,
             kout_hbm, oout_hbm, sout_hbm,
             kidx_v, krows_v, oidx_v, orows_v, sidx_v, srows_v, sem):
        wid = lax.axis_index("s") * NC + lax.axis_index("c")

        sbase = wid * SROWS
        for i in range(2):
            pltpu.sync_copy(scat_hbm.at[pl.ds(i * B + sbase, SROWS)], sidx_v)
            if i:
                for v in range(SROWS // 16):
                    sl = pl.ds(v * 16, 16)
                    sidx_v[sl] = sidx_v[sl] + i * SV
            pltpu.async_copy(stab_hbm.at[sidx_v], srows_v, sem).wait()
            pltpu.sync_copy(srows_v, sout_hbm.at[pl.ds(i * B + sbase, SROWS)])

        kv = wid // (NW // 4)
        koff = kv * KV
        rbase = (wid % (NW // 4)) * KNCH

        def kblock(jb, carry):
            c0 = jb * KSLOT
            for u in range(KSLOT):
                src = (wid * KNCH + c0 + u) * KCH
                pltpu.sync_copy(kcat_hbm.at[pl.ds(src, KCH)], kidx_v.at[u])
                for q in range(KCH // 16):
                    sl = pl.ds(q * 16, 16)
                    kidx_v[u, sl] = kidx_v[u, sl] + koff
            descs = [
                pltpu.async_copy(ktab_hbm.at[kidx_v.at[u]], krows_v.at[u], sem)
                for u in range(KSLOT)
            ]
            for d in descs:
                d.wait()
            for u in range(KSLOT):
                r = rbase + c0 + u
                t = r // (B // KCH)
                bc = r % (B // KCH)
                dst = (t * 12 + kv) * B + bc * KCH
                pltpu.sync_copy(krows_v.at[u], kout_hbm.at[pl.ds(dst, KCH)])
            return carry

        lax.fori_loop(0, KNCH // KSLOT, kblock, 0)

        def oblock(c, carry):
            base = wid * ROWS_W + c * OCH
            for g in range(2):
                pltpu.sync_copy(
                    ocat_hbm.at[pl.ds(g * BT + base, OCH)], oidx_v.at[g])
            for q in range(OCH // 16):
                sl = pl.ds(q * 16, 16)
                oidx_v[1, sl] = oidx_v[1, sl] + OV
            descs = [
                pltpu.async_copy(otab_hbm.at[oidx_v.at[g]], orows_v.at[g], sem)
                for g in range(2)
            ]
            for d in descs:
                d.wait()
            descs = [
                pltpu.async_copy(
                    orows_v.at[g],
                    oout_hbm.at[pl.ds(base, OCH), pl.ds(g * H, H)], sem)
                for g in range(2)
            ]
            for d in descs:
                d.wait()
            return carry

        lax.fori_loop(0, ONCH, oblock, 0)

    return body(kcat_f, ocat_f, scat_f, k_tab, o_tab, s_tab)


def _known_cont_body(c_ref, vec_ref, bias_ref, alias_ref, out_ref):
    j = pl.program_id(1)
    c = c_ref[0]
    vec = vec_ref[...]
    bias = bias_ref[...]
    for jj in range(2):

        @pl.when(j == jj)
        def _():
            for u in range(4):
                cv = 4 * jj + u
                out_ref[u * B:(u + 1) * B, :] = jnp.dot(
                    c[:, cv:cv + 1], vec[cv:cv + 1, :],
                    preferred_element_type=jnp.float32) + bias[cv:cv + 1, :]


def _known_cont_fill(cont_tb, vec, bias, cat_buf):
    return pl.pallas_call(
        _known_cont_body,
        grid=(T, 2),
        in_specs=[
            pl.BlockSpec((1, B, 8), lambda t, j: (t, 0, 0)),
            pl.BlockSpec((8, H), lambda t, j: (0, 0)),
            pl.BlockSpec((8, H), lambda t, j: (0, 0)),
            pl.BlockSpec((8, H), lambda t, j: (0, 0)),
        ],
        out_specs=pl.BlockSpec((4 * B, H), lambda t, j: (3 * t + 1 + j, 0)),
        out_shape=jax.ShapeDtypeStruct((T * 12 * B, H), jnp.float32),
        input_output_aliases={3: 0},
    )(cont_tb, vec, bias, cat_buf)


OR_CH = 2560


def _obs_cont_body(c_ref, vec_ref, bias_ref, alias_ref, out_ref):
    j = pl.program_id(1)
    c = c_ref[...]
    vec = vec_ref[...]
    bias = bias_ref[...]
    for jj in range(3):

        @pl.when(j == jj)
        def _():
            for u in range(2):
                cv = 2 * jj + u
                out_ref[:, u * H:(u + 1) * H] = jnp.dot(
                    c[:, cv:cv + 1], vec[cv:cv + 1, :],
                    preferred_element_type=jnp.float32) + bias[cv:cv + 1, :]


def _obs_cont_fill(cont2, vec, bias, cat_buf):
    return pl.pallas_call(
        _obs_cont_body,
        grid=(BT // OR_CH, 3),
        in_specs=[
            pl.BlockSpec((OR_CH, 6), lambda i, j: (i, 0)),
            pl.BlockSpec((6, H), lambda i, j: (0, 0)),
            pl.BlockSpec((6, H), lambda i, j: (0, 0)),
            pl.BlockSpec((8, 128), lambda i, j: (0, 0)),
        ],
        out_specs=pl.BlockSpec((OR_CH, 2 * H), lambda i, j: (i, j + 1)),
        out_shape=jax.ShapeDtypeStruct((BT, 8 * H), jnp.float32),
        input_output_aliases={3: 0},
    )(cont2, vec, bias, cat_buf)


def _static_cont_body(c_ref, vec_ref, bias_ref, alias_ref, out_ref):
    v = pl.program_id(0)
    c = c_ref[...]
    vec = vec_ref[...]
    bias = bias_ref[...]
    for vv in range(4):

        @pl.when(v == vv)
        def _():
            out_ref[...] = jnp.dot(
                c[:, vv:vv + 1], vec[vv:vv + 1, :],
                preferred_element_type=jnp.float32) + bias[vv:vv + 1, :]


def _static_cont_fill(cont2, vec, bias, cat_buf):
    return pl.pallas_call(
        _static_cont_body,
        grid=(4,),
        in_specs=[
            pl.BlockSpec((B, 4), lambda v: (0, 0)),
            pl.BlockSpec((4, H), lambda v: (0, 0)),
            pl.BlockSpec((4, H), lambda v: (0, 0)),
            pl.BlockSpec((8, H), lambda v: (0, 0)),
        ],
        out_specs=pl.BlockSpec((B, H), lambda v: (v + 2, 0)),
        out_shape=jax.ShapeDtypeStruct((6 * B, H), jnp.float32),
        input_output_aliases={3: 0},
    )(cont2, vec, bias, cat_buf)


GBT = 32


def _tgt_body(c_ref, vec_ref, bias_ref, out_ref):
    out_ref[:, :, 0, :] = (
        c_ref[...] * vec_ref[...][0][None, :] + bias_ref[...][0][None, :]
    ).reshape(GBT, T, H)


def _tgt_fill(cont2, vec, bias):
    return pl.pallas_call(
        _tgt_body,
        grid=(B // GBT,),
        in_specs=[
            pl.BlockSpec((GBT * T, 1), lambda i: (i, 0)),
            pl.BlockSpec((1, H), lambda i: (0, 0)),
            pl.BlockSpec((1, H), lambda i: (0, 0)),
        ],
        out_specs=pl.BlockSpec((GBT, T, 1, H), lambda i: (i, 0, 0, 0)),
        out_shape=jax.ShapeDtypeStruct((B, T, 1, H), jnp.float32),
    )(cont2, vec, bias)


def kernel(s_cat, s_cont, k_cat, k_cont, o_cat, o_cont, target,
           s_cat_tables, k_cat_tables, o_cat_tables,
           s_cont_vec, s_cont_bias, k_cont_vec, k_cont_bias,
           o_cont_vec, o_cont_bias, tgt_vec, tgt_bias):
    kcat_f = jnp.transpose(k_cat, (2, 1, 0)).reshape(-1)
    ocat_f = o_cat.reshape(BT, 2).T.reshape(-1)
    scat_f = s_cat[:, 0, :].T.reshape(-1)
    k_tab = k_cat_tables.reshape(4 * KV, H)
    o_tab = o_cat_tables.reshape(2 * OV, H)
    s_tab = s_cat_tables.reshape(2 * SV, H)

    kc_tb = jnp.transpose(k_cont, (1, 0, 2))
    t_full = _tgt_fill(target.reshape(BT, 1), tgt_vec, tgt_bias)

    kbuf, obuf, sbuf = _sc_gather(kcat_f, ocat_f, scat_f, k_tab, o_tab, s_tab)

    kbuf = _known_cont_fill(kc_tb, k_cont_vec, k_cont_bias, kbuf)
    obuf = _obs_cont_fill(o_cont.reshape(BT, 6),
                          o_cont_vec, o_cont_bias, obuf)
    sbuf = _static_cont_fill(s_cont[:, 0, :],
                             s_cont_vec, s_cont_bias, sbuf)

    k_full = jnp.transpose(kbuf.reshape(T, 12, B, H), (2, 0, 1, 3))
    o_full = obuf.reshape(B, T, 8, H)
    s_full = jnp.transpose(sbuf.reshape(6, B, H), (1, 0, 2))
    return (s_full, k_full, o_full, t_full)
